# Initial kernel scaffold; baseline (speedup 1.0000x reference)
import jax
import jax.numpy as jnp
from jax.experimental import pallas as pl


def kernel(x, edge_index, W1, b1, W2, b2):
    raise NotImplementedError("write your pallas kernel here")


# speedup vs baseline: 13.6711x; 13.6711x over previous
import functools

import jax
import jax.numpy as jnp
from jax import lax
from jax.experimental import pallas as pl
from jax.experimental.pallas import tpu as pltpu
from jax.experimental.pallas import tpu_sc as plsc

N = 10000
D = 128
E = 320000
NC = 2
NS = 16
NW = NC * NS
CHUNK = 128
NBLK = 79
N_PAD = NBLK * 128
PAD_ROWS = N_PAD - N
EPT = N_PAD
NCHUNK = EPT // CHUNK
E_PAD = NW * EPT
HPT = 640
N_HIST = NS * HPT
RPT = N_PAD // NS

_mesh = plsc.VectorSubcoreMesh(
    core_axis_name="c", subcore_axis_name="s", num_cores=NC, num_subcores=NS
)


@functools.partial(
    pl.kernel,
    out_type=jax.ShapeDtypeStruct((NC, N_HIST), jnp.float32),
    mesh=_mesh,
    scratch_types=[
        pltpu.VMEM_SHARED((N_HIST,), jnp.float32),
        pltpu.VMEM((HPT,), jnp.float32),
        pltpu.VMEM((CHUNK,), jnp.float32),
        pltpu.VMEM((CHUNK,), jnp.int32),
    ],
)
def _deg_kernel(dst_hbm, out_hbm, hist, zbuf, ones, idx):
    c = lax.axis_index("c")
    s = lax.axis_index("s")
    for i in range(HPT // 16):
        zbuf[pl.ds(i * 16, 16)] = jnp.zeros((16,), jnp.float32)
    for i in range(CHUNK // 16):
        ones[pl.ds(i * 16, 16)] = jnp.ones((16,), jnp.float32)
    pltpu.sync_copy(zbuf, hist.at[pl.ds(s * HPT, HPT)])
    plsc.subcore_barrier()

    def body(j, carry):
        pltpu.sync_copy(dst_hbm.at[c, s, j], idx)
        pltpu.sync_copy(ones, hist.at[idx], add=True)
        return carry

    lax.fori_loop(0, NCHUNK, body, 0)
    plsc.subcore_barrier()
    pltpu.sync_copy(hist.at[pl.ds(s * HPT, HPT)], out_hbm.at[c, pl.ds(s * HPT, HPT)])


@functools.partial(
    pl.kernel,
    out_type=jax.ShapeDtypeStruct((NC, N_PAD, D), jnp.float32),
    mesh=_mesh,
    scratch_types=[
        pltpu.VMEM_SHARED((N_PAD, D), jnp.float32),
        pltpu.VMEM((CHUNK, D), jnp.float32),
        pltpu.VMEM((CHUNK, D), jnp.float32),
        pltpu.VMEM((CHUNK,), jnp.int32),
        pltpu.VMEM((CHUNK,), jnp.int32),
        pltpu.SemaphoreType.DMA,
    ],
)
def _scatter_kernel(g_hbm, src_hbm, dst_hbm, out_hbm, acc, rows, zbuf, sidx, didx, sem):
    c = lax.axis_index("c")
    s = lax.axis_index("s")

    def zrow(i, carry):
        for k in range(D // 16):
            zbuf[i, pl.ds(k * 16, 16)] = jnp.zeros((16,), jnp.float32)
        return carry

    lax.fori_loop(0, CHUNK, zrow, 0)
    base = s * RPT
    for r in range(RPT // CHUNK):
        pltpu.sync_copy(zbuf, acc.at[pl.ds(base + r * CHUNK, CHUNK)])
    rem = RPT % CHUNK
    pltpu.sync_copy(
        zbuf.at[pl.ds(0, rem)], acc.at[pl.ds(base + RPT - rem, rem)]
    )
    plsc.subcore_barrier()

    def body(j, carry):
        pltpu.sync_copy(src_hbm.at[c, s, j], sidx)
        pltpu.sync_copy(dst_hbm.at[c, s, j], didx)
        pltpu.async_copy(g_hbm.at[sidx], rows, sem).wait()
        pltpu.sync_copy(rows, acc.at[didx], add=True)
        return carry

    lax.fori_loop(0, NCHUNK, body, 0)
    plsc.subcore_barrier()
    for r in range(RPT // CHUNK):
        sl = pl.ds(base + r * CHUNK, CHUNK)
        pltpu.sync_copy(acc.at[sl], out_hbm.at[c, sl])
    sl = pl.ds(base + RPT - rem, rem)
    pltpu.sync_copy(acc.at[sl], out_hbm.at[c, sl])


def _pre_body(deg_ref, x_ref, w_ref, g_ref, dinv_ref):
    dinv = lax.rsqrt(deg_ref[...] + 1.0)
    g_ref[...] = dinv * jnp.dot(
        x_ref[...], w_ref[...], preferred_element_type=jnp.float32
    )
    dinv_ref[...] = dinv


def _mid_body(a0_ref, a1_ref, g_ref, dinv_ref, b_ref, w_ref, out_ref):
    dv = dinv_ref[...]
    h = jnp.maximum(dv * (a0_ref[...] + a1_ref[...] + g_ref[...]) + b_ref[...], 0.0)
    out_ref[...] = dv * jnp.dot(h, w_ref[...], preferred_element_type=jnp.float32)


def _post_body(a0_ref, a1_ref, g_ref, dinv_ref, b_ref, out_ref):
    out_ref[...] = jnp.maximum(
        dinv_ref[...] * (a0_ref[...] + a1_ref[...] + g_ref[...]) + b_ref[...], 0.0
    )


def _row_spec(w):
    return pl.BlockSpec((CHUNK, w), lambda i: (i, 0))


def _full_spec(h, w):
    return pl.BlockSpec((h, w), lambda i: (0, 0))


_f32 = jnp.float32

_pre_call = pl.pallas_call(
    _pre_body,
    grid=(NBLK,),
    in_specs=[_row_spec(1), _row_spec(D), _full_spec(D, D)],
    out_specs=[_row_spec(D), _row_spec(1)],
    out_shape=[
        jax.ShapeDtypeStruct((N_PAD, D), _f32),
        jax.ShapeDtypeStruct((N_PAD, 1), _f32),
    ],
)

_mid_call = pl.pallas_call(
    _mid_body,
    grid=(NBLK,),
    in_specs=[
        _row_spec(D),
        _row_spec(D),
        _row_spec(D),
        _row_spec(1),
        _full_spec(1, D),
        _full_spec(D, D),
    ],
    out_specs=_row_spec(D),
    out_shape=jax.ShapeDtypeStruct((N_PAD, D), _f32),
)

_post_call = pl.pallas_call(
    _post_body,
    grid=(NBLK,),
    in_specs=[_row_spec(D), _row_spec(D), _row_spec(D), _row_spec(1), _full_spec(1, D)],
    out_specs=_row_spec(D),
    out_shape=jax.ShapeDtypeStruct((N_PAD, D), _f32),
)


def kernel(x, edge_index, W1, b1, W2, b2):
    x_pad = jnp.pad(x, ((0, PAD_ROWS), (0, 0)))
    pad_idx = (N + (jnp.arange(E_PAD - E, dtype=jnp.int32) % PAD_ROWS)).astype(
        jnp.int32
    )
    src = jnp.concatenate([edge_index[0], pad_idx]).reshape(NC, NS, NCHUNK, CHUNK)
    dst = jnp.concatenate([edge_index[1], pad_idx]).reshape(NC, NS, NCHUNK, CHUNK)

    deg_parts = _deg_kernel(dst)
    degsum_col = (deg_parts[0, :N_PAD] + deg_parts[1, :N_PAD])[:, None]

    g1, dinv = _pre_call(degsum_col, x_pad, W1)
    acc1 = _scatter_kernel(g1, src, dst)
    g2 = _mid_call(acc1[0], acc1[1], g1, dinv, b1[None, :], W2)
    acc2 = _scatter_kernel(g2, src, dst)
    out = _post_call(acc2[0], acc2[1], g2, dinv, b2[None, :])
    return out[:N]


# speedup vs baseline: 18.8647x; 1.3799x over previous
import functools

import jax
import jax.numpy as jnp
from jax import lax
from jax.experimental import pallas as pl
from jax.experimental.pallas import tpu as pltpu
from jax.experimental.pallas import tpu_sc as plsc

N = 10000
D = 128
HD = D // 2
E = 320000
NC = 2
NS = 16
NW = NC * NS
CHUNK = 128
NBLK = 79
N_PAD = NBLK * 128
PAD_ROWS = N_PAD - N
E_PAD = NW * 80 * CHUNK
NCH_D = 80
NCH_S = E_PAD // (NS * CHUNK)
HPT = 640
N_HIST = NS * HPT
RPT = N_PAD // NS
NBUF = 4

_mesh = plsc.VectorSubcoreMesh(
    core_axis_name="c", subcore_axis_name="s", num_cores=NC, num_subcores=NS
)


@functools.partial(
    pl.kernel,
    out_type=jax.ShapeDtypeStruct((NC, N_HIST), jnp.float32),
    mesh=_mesh,
    scratch_types=[
        pltpu.VMEM_SHARED((N_HIST,), jnp.float32),
        pltpu.VMEM((HPT,), jnp.float32),
        pltpu.VMEM((CHUNK,), jnp.float32),
        pltpu.VMEM((NCH_D, CHUNK), jnp.int32),
        pltpu.SemaphoreType.DMA,
    ],
)
def _deg_kernel(dst_hbm, out_hbm, hist, zbuf, ones, idx, sem):
    c = lax.axis_index("c")
    s = lax.axis_index("s")
    for i in range(HPT // 16):
        zbuf[pl.ds(i * 16, 16)] = jnp.zeros((16,), jnp.float32)
    for i in range(CHUNK // 16):
        ones[pl.ds(i * 16, 16)] = jnp.ones((16,), jnp.float32)
    pltpu.sync_copy(dst_hbm.at[c, s], idx)
    pltpu.sync_copy(zbuf, hist.at[pl.ds(s * HPT, HPT)])
    plsc.subcore_barrier()

    def body(j, carry):
        pltpu.async_copy(ones, hist.at[idx.at[j]], sem, add=True)
        return carry

    lax.fori_loop(0, NCH_D, body, 0)
    pltpu.make_async_copy(dst_hbm.at[c, s], idx, sem).wait()
    plsc.subcore_barrier()
    pltpu.sync_copy(hist.at[pl.ds(s * HPT, HPT)], out_hbm.at[c, pl.ds(s * HPT, HPT)])


def _scatter_body(g_hbm, sd_hbm, out_hbm, acc, sd, rows, gsems, ssems):
    c = lax.axis_index("c")
    s = lax.axis_index("s")

    def zrow(i, carry):
        for k in range(HD // 16):
            rows[0][i, pl.ds(k * 16, 16)] = jnp.zeros((16,), jnp.float32)
        return carry

    lax.fori_loop(0, CHUNK, zrow, 0)
    base = s * RPT
    rem = RPT % CHUNK
    for r in range(RPT // CHUNK):
        pltpu.sync_copy(rows[0], acc.at[pl.ds(base + r * CHUNK, CHUNK)])
    pltpu.sync_copy(rows[0].at[pl.ds(0, rem)], acc.at[pl.ds(base + RPT - rem, rem)])

    pltpu.sync_copy(sd_hbm.at[c, s], sd)

    for b in range(NBUF):
        pltpu.async_copy(g_hbm.at[sd.at[b, 0]], rows[b], gsems[b])
    plsc.subcore_barrier()

    def body(i, carry):
        j0 = i * NBUF
        for b in range(NBUF):
            pltpu.make_async_copy(g_hbm.at[sd.at[j0 + b, 0]], rows[b], gsems[b]).wait()
            pltpu.async_copy(rows[b], acc.at[sd.at[j0 + b, 1]], ssems[b], add=True)
        for b in range(NBUF):
            pltpu.make_async_copy(rows[b], acc.at[sd.at[j0 + b, 1]], ssems[b]).wait()
            pltpu.async_copy(g_hbm.at[sd.at[j0 + NBUF + b, 0]], rows[b], gsems[b])
        return carry

    lax.fori_loop(0, NCH_S // NBUF - 1, body, 0)
    for b in range(NBUF):
        j = NCH_S - NBUF + b
        pltpu.make_async_copy(g_hbm.at[sd.at[j, 0]], rows[b], gsems[b]).wait()
        pltpu.async_copy(rows[b], acc.at[sd.at[j, 1]], ssems[b], add=True)
    for b in range(NBUF):
        j = NCH_S - NBUF + b
        pltpu.make_async_copy(rows[b], acc.at[sd.at[j, 1]], ssems[b]).wait()
    plsc.subcore_barrier()

    for r in range(RPT // CHUNK):
        sl = pl.ds(base + r * CHUNK, CHUNK)
        pltpu.sync_copy(acc.at[sl], out_hbm.at[sl, c])
    sl = pl.ds(base + RPT - rem, rem)
    pltpu.sync_copy(acc.at[sl], out_hbm.at[sl, c])


_scatter_kernel = pl.kernel(
    _scatter_body,
    out_type=jax.ShapeDtypeStruct((N_PAD, NC, HD), jnp.float32),
    mesh=_mesh,
    compiler_params=pltpu.CompilerParams(use_tc_tiling_on_sc=False),
    scratch_types=[
        pltpu.VMEM_SHARED((N_PAD, HD), jnp.float32),
        pltpu.VMEM((NCH_S, 2, CHUNK), jnp.int32),
        [pltpu.VMEM((CHUNK, HD), jnp.float32)] * NBUF,
        [pltpu.SemaphoreType.DMA] * NBUF,
        [pltpu.SemaphoreType.DMA] * NBUF,
    ],
)


def _mm_body(x_ref, w_ref, h_ref):
    h_ref[...] = jnp.dot(x_ref[...], w_ref[...], preferred_element_type=jnp.float32)


def _scale_body(deg_ref, h_ref, g_ref, dinv_ref):
    dinv = lax.rsqrt(deg_ref[...] + 1.0)
    g_ref[...] = dinv * h_ref[...]
    dinv_ref[...] = dinv


def _mid_body(s_ref, g_ref, dinv_ref, b_ref, w_ref, out_ref):
    dv = dinv_ref[...]
    h = jnp.maximum(dv * (s_ref[...] + g_ref[...]) + b_ref[...], 0.0)
    out_ref[...] = dv * jnp.dot(h, w_ref[...], preferred_element_type=jnp.float32)


def _post_body(s_ref, g_ref, dinv_ref, b_ref, out_ref):
    out_ref[...] = jnp.maximum(
        dinv_ref[...] * (s_ref[...] + g_ref[...]) + b_ref[...], 0.0
    )


def _row_spec(w):
    return pl.BlockSpec((CHUNK, w), lambda i: (i, 0))


def _full_spec(h, w):
    return pl.BlockSpec((h, w), lambda i: (0, 0))


_f32 = jnp.float32

_mm_call = pl.pallas_call(
    _mm_body,
    grid=(NBLK,),
    in_specs=[_row_spec(D), _full_spec(D, D)],
    out_specs=_row_spec(D),
    out_shape=jax.ShapeDtypeStruct((N_PAD, D), _f32),
)

_scale_call = pl.pallas_call(
    _scale_body,
    grid=(NBLK,),
    in_specs=[_row_spec(1), _row_spec(D)],
    out_specs=[_row_spec(D), _row_spec(1)],
    out_shape=[
        jax.ShapeDtypeStruct((N_PAD, D), _f32),
        jax.ShapeDtypeStruct((N_PAD, 1), _f32),
    ],
)

_mid_call = pl.pallas_call(
    _mid_body,
    grid=(NBLK,),
    in_specs=[
        _row_spec(D),
        _row_spec(D),
        _row_spec(1),
        _full_spec(1, D),
        _full_spec(D, D),
    ],
    out_specs=_row_spec(D),
    out_shape=jax.ShapeDtypeStruct((N_PAD, D), _f32),
)

_post_call = pl.pallas_call(
    _post_body,
    grid=(NBLK,),
    in_specs=[_row_spec(D), _row_spec(D), _row_spec(1), _full_spec(1, D)],
    out_specs=_row_spec(D),
    out_shape=jax.ShapeDtypeStruct((N_PAD, D), _f32),
)


def kernel(x, edge_index, W1, b1, W2, b2):
    x_pad = jnp.pad(x, ((0, PAD_ROWS), (0, 0)))
    pad_idx = (N + (jnp.arange(E_PAD - E, dtype=jnp.int32) % PAD_ROWS)).astype(
        jnp.int32
    )
    src = jnp.concatenate([edge_index[0], pad_idx])
    dst = jnp.concatenate([edge_index[1], pad_idx])
    src_t = src.reshape(NS, NCH_S, 1, CHUNK)
    dst_t = dst.reshape(NS, NCH_S, 1, CHUNK)
    sd_c = [
        jnp.concatenate([2 * src_t + c, dst_t], axis=2)[None] for c in range(NC)
    ]
    sd = jnp.concatenate(sd_c, axis=0)

    h1 = _mm_call(x_pad, W1)
    deg_parts = _deg_kernel(dst.reshape(NC, NS, NCH_D, CHUNK))
    degsum_col = (deg_parts[0, :N_PAD] + deg_parts[1, :N_PAD])[:, None]

    g1, dinv = _scale_call(degsum_col, h1)
    s1 = _scatter_kernel(g1.reshape(NC * N_PAD, HD), sd).reshape(N_PAD, D)
    g2 = _mid_call(s1, g1, dinv, b1[None, :], W2)
    s2 = _scatter_kernel(g2.reshape(NC * N_PAD, HD), sd).reshape(N_PAD, D)
    out = _post_call(s2, g2, dinv, b2[None, :])
    return out[:N]


# speedup vs baseline: 26.6303x; 1.4116x over previous
import functools

import jax
import jax.numpy as jnp
from jax import lax
from jax.experimental import pallas as pl
from jax.experimental.pallas import tpu as pltpu
from jax.experimental.pallas import tpu_sc as plsc

N = 10000
D = 128
HD = D // 2
E = 320000
NC = 2
NS = 16
NW = NC * NS
CHUNK = 128
NBLK = 79
N_PAD = NBLK * 128
PAD_ROWS = N_PAD - N
E_PAD = NW * 80 * CHUNK
NCH_D = 80
NCH_S = E_PAD // (NS * CHUNK)
HPT = 640
N_HIST = NS * HPT
RPT = N_PAD // NS
NBUF = 4

_mesh = plsc.VectorSubcoreMesh(
    core_axis_name="c", subcore_axis_name="s", num_cores=NC, num_subcores=NS
)


@functools.partial(
    pl.kernel,
    out_type=jax.ShapeDtypeStruct((NC, N_HIST), jnp.float32),
    mesh=_mesh,
    scratch_types=[
        pltpu.VMEM_SHARED((N_HIST,), jnp.float32),
        pltpu.VMEM((HPT,), jnp.float32),
        pltpu.VMEM((CHUNK,), jnp.float32),
        pltpu.VMEM((NCH_D, CHUNK), jnp.int32),
        pltpu.SemaphoreType.DMA,
    ],
)
def _deg_kernel(dst_hbm, out_hbm, hist, zbuf, ones, idx, sem):
    c = lax.axis_index("c")
    s = lax.axis_index("s")
    for i in range(HPT // 16):
        zbuf[pl.ds(i * 16, 16)] = jnp.zeros((16,), jnp.float32)
    for i in range(CHUNK // 16):
        ones[pl.ds(i * 16, 16)] = jnp.ones((16,), jnp.float32)
    pltpu.sync_copy(dst_hbm.at[c, s], idx)
    pltpu.sync_copy(zbuf, hist.at[pl.ds(s * HPT, HPT)])
    plsc.subcore_barrier()

    def body(j, carry):
        pltpu.async_copy(ones, hist.at[idx.at[j]], sem, add=True)
        return carry

    lax.fori_loop(0, NCH_D, body, 0)
    pltpu.make_async_copy(dst_hbm.at[c, s], idx, sem).wait()
    plsc.subcore_barrier()
    pltpu.sync_copy(hist.at[pl.ds(s * HPT, HPT)], out_hbm.at[c, pl.ds(s * HPT, HPT)])


def _scatter_body(g_hbm, sd_hbm, out_hbm, acc, sd, rows, gsems, ssems):
    c = lax.axis_index("c")
    s = lax.axis_index("s")

    def zrow(i, carry):
        for k in range(HD // 16):
            rows[0][i, pl.ds(k * 16, 16)] = jnp.zeros((16,), jnp.float32)
        return carry

    lax.fori_loop(0, CHUNK, zrow, 0)
    base = s * RPT
    rem = RPT % CHUNK
    for r in range(RPT // CHUNK):
        pltpu.sync_copy(rows[0], acc.at[pl.ds(base + r * CHUNK, CHUNK)])
    pltpu.sync_copy(rows[0].at[pl.ds(0, rem)], acc.at[pl.ds(base + RPT - rem, rem)])

    pltpu.sync_copy(sd_hbm.at[c, s], sd)

    for b in range(NBUF):
        pltpu.async_copy(g_hbm.at[sd.at[b, 0]], rows[b], gsems[b])
    plsc.subcore_barrier()

    def body(i, carry):
        j0 = i * NBUF
        for b in range(NBUF):
            pltpu.make_async_copy(g_hbm.at[sd.at[j0 + b, 0]], rows[b], gsems[b]).wait()
            pltpu.async_copy(rows[b], acc.at[sd.at[j0 + b, 1]], ssems[b], add=True)
        for b in range(NBUF):
            pltpu.make_async_copy(rows[b], acc.at[sd.at[j0 + b, 1]], ssems[b]).wait()
            pltpu.async_copy(g_hbm.at[sd.at[j0 + NBUF + b, 0]], rows[b], gsems[b])
        return carry

    lax.fori_loop(0, NCH_S // NBUF - 1, body, 0)
    for b in range(NBUF):
        j = NCH_S - NBUF + b
        pltpu.make_async_copy(g_hbm.at[sd.at[j, 0]], rows[b], gsems[b]).wait()
        pltpu.async_copy(rows[b], acc.at[sd.at[j, 1]], ssems[b], add=True)
    for b in range(NBUF):
        j = NCH_S - NBUF + b
        pltpu.make_async_copy(rows[b], acc.at[sd.at[j, 1]], ssems[b]).wait()
    plsc.subcore_barrier()

    for r in range(RPT // CHUNK):
        sl = pl.ds(base + r * CHUNK, CHUNK)
        pltpu.sync_copy(acc.at[sl], out_hbm.at[sl, c])
    sl = pl.ds(base + RPT - rem, rem)
    pltpu.sync_copy(acc.at[sl], out_hbm.at[sl, c])


_scatter_kernel = pl.kernel(
    _scatter_body,
    out_type=jax.ShapeDtypeStruct((N_PAD, NC, HD), jnp.float32),
    mesh=_mesh,
    compiler_params=pltpu.CompilerParams(use_tc_tiling_on_sc=False),
    scratch_types=[
        pltpu.VMEM_SHARED((N_PAD, HD), jnp.float32),
        pltpu.VMEM((NCH_S, 2, CHUNK), jnp.int32),
        [pltpu.VMEM((CHUNK, HD), jnp.float32)] * NBUF,
        [pltpu.SemaphoreType.DMA] * NBUF,
        [pltpu.SemaphoreType.DMA] * NBUF,
    ],
)


def _pre_body(deg_ref, x_ref, w_ref, g_ref, dinv_ref):
    dinv = lax.rsqrt(deg_ref[...] + 1.0)
    g_ref[...] = dinv * jnp.dot(
        x_ref[...], w_ref[...], preferred_element_type=jnp.float32
    )
    dinv_ref[...] = dinv


def _mid_body(s_ref, g_ref, dinv_ref, b_ref, w_ref, out_ref):
    dv = dinv_ref[...]
    h = jnp.maximum(dv * (s_ref[...] + g_ref[...]) + b_ref[...], 0.0)
    out_ref[...] = dv * jnp.dot(h, w_ref[...], preferred_element_type=jnp.float32)


def _post_body(s_ref, g_ref, dinv_ref, b_ref, out_ref):
    out_ref[...] = jnp.maximum(
        dinv_ref[...] * (s_ref[...] + g_ref[...]) + b_ref[...], 0.0
    )


RBLK = 1264
NBLK_TC = N_PAD // RBLK


def _row_spec(w):
    return pl.BlockSpec((RBLK, w), lambda i: (i, 0))


def _full_spec(h, w):
    return pl.BlockSpec((h, w), lambda i: (0, 0))


_f32 = jnp.float32

_pre_call = pl.pallas_call(
    _pre_body,
    grid=(NBLK_TC,),
    in_specs=[_row_spec(1), _row_spec(D), _full_spec(D, D)],
    out_specs=[_row_spec(D), _row_spec(1)],
    out_shape=[
        jax.ShapeDtypeStruct((N_PAD, D), _f32),
        jax.ShapeDtypeStruct((N_PAD, 1), _f32),
    ],
)

_mid_call = pl.pallas_call(
    _mid_body,
    grid=(NBLK_TC,),
    in_specs=[
        _row_spec(D),
        _row_spec(D),
        _row_spec(1),
        _full_spec(1, D),
        _full_spec(D, D),
    ],
    out_specs=_row_spec(D),
    out_shape=jax.ShapeDtypeStruct((N_PAD, D), _f32),
)

_post_call = pl.pallas_call(
    _post_body,
    grid=(NBLK_TC,),
    in_specs=[_row_spec(D), _row_spec(D), _row_spec(1), _full_spec(1, D)],
    out_specs=_row_spec(D),
    out_shape=jax.ShapeDtypeStruct((N_PAD, D), _f32),
)


def kernel(x, edge_index, W1, b1, W2, b2):
    x_pad = jnp.pad(x, ((0, PAD_ROWS), (0, 0)))
    pad_idx = (N + (jnp.arange(E_PAD - E, dtype=jnp.int32) % PAD_ROWS)).astype(
        jnp.int32
    )
    src = jnp.concatenate([edge_index[0], pad_idx])
    dst = jnp.concatenate([edge_index[1], pad_idx])
    src_t = src.reshape(NS, NCH_S, 1, CHUNK)
    dst_t = dst.reshape(NS, NCH_S, 1, CHUNK)
    sd_c = [
        jnp.concatenate([2 * src_t + c, dst_t], axis=2)[None] for c in range(NC)
    ]
    sd = jnp.concatenate(sd_c, axis=0)

    deg_parts = _deg_kernel(dst.reshape(NC, NS, NCH_D, CHUNK))
    degsum_col = (deg_parts[0, :N_PAD] + deg_parts[1, :N_PAD])[:, None]

    g1, dinv = _pre_call(degsum_col, x_pad, W1)
    s1 = _scatter_kernel(g1.reshape(NC * N_PAD, HD), sd).reshape(N_PAD, D)
    g2 = _mid_call(s1, g1, dinv, b1[None, :], W2)
    s2 = _scatter_kernel(g2.reshape(NC * N_PAD, HD), sd).reshape(N_PAD, D)
    out = _post_call(s2, g2, dinv, b2[None, :])
    return out[:N]
